# baseline (speedup 1.0000x reference)
import functools

import jax
import jax.numpy as jnp
from jax import lax
from jax.experimental import pallas as pl
from jax.experimental.pallas import tpu as pltpu
from jax.experimental.pallas import tpu_sc as plsc

N, D_IN, D, A, B, NACT = 16384, 64, 256, 4096, 16, 16
TILE = 2048


def _tc_body(e_ref, bi_ref, w1_ref, b1_ref, w2_ref, b2_ref, wh_ref, bh_ref,
             wa_ref, ba_ref, lp_ref, ent_ref, aux_ref, seg_acc, cnt_acc):
    i = pl.program_id(0)

    emb = jnp.maximum(
        jnp.dot(e_ref[...], w1_ref[...], preferred_element_type=jnp.float32)
        + b1_ref[...], 0.0)
    x = jnp.maximum(
        jnp.dot(emb, w2_ref[...], preferred_element_type=jnp.float32)
        + b2_ref[...], 0.0)

    logits = jnp.dot(x, wh_ref[...], preferred_element_type=jnp.float32) + bh_ref[...]
    m = jnp.max(logits, axis=1, keepdims=True)
    t = logits - m
    s = jnp.exp(t)
    ssum = jnp.sum(s, axis=1, keepdims=True)
    logsum = jnp.log(ssum)
    lp_ref[...] = t - logsum
    ent_ref[...] = logsum - jnp.sum(s * t, axis=1, keepdims=True) / ssum

    seg_ids = lax.broadcasted_iota(jnp.int32, (TILE, B), 1)
    onehot = (bi_ref[...] == seg_ids).astype(jnp.float32)
    x_hi = x.astype(jnp.bfloat16).astype(jnp.float32)
    x_lo = x - x_hi
    dims = (((0,), (0,)), ((), ()))
    seg = (lax.dot_general(onehot, x_hi, dims,
                           preferred_element_type=jnp.float32)
           + lax.dot_general(onehot, x_lo, dims,
                             preferred_element_type=jnp.float32))
    cnt = lax.dot_general(onehot, jnp.ones((TILE, 1), jnp.float32), dims,
                          preferred_element_type=jnp.float32)

    @pl.when(i == 0)
    def _init():
        seg_acc[...] = seg
        cnt_acc[...] = cnt

    @pl.when(i > 0)
    def _accum():
        seg_acc[...] += seg
        cnt_acc[...] += cnt

    @pl.when(i == pl.num_programs(0) - 1)
    def _finalize():
        pooled = seg_acc[...] / jnp.maximum(cnt_acc[...], 1.0)
        pooled_b = pooled.astype(jnp.bfloat16).astype(jnp.float32)
        wa_b = wa_ref[...].astype(jnp.bfloat16).astype(jnp.float32)
        aux_ref[...] = (jnp.sum(pooled_b * wa_b, axis=1, keepdims=True)
                        + ba_ref[...])


def _tc_stage(entities, batch_index_col, W_emb, b_emb, W_bb, b_bb,
              W_head, b_head, W_aux, b_aux):
    grid = N // TILE
    rep = lambda i: (0, 0)
    return pl.pallas_call(
        _tc_body,
        grid=(grid,),
        in_specs=[
            pl.BlockSpec((TILE, D_IN), lambda i: (i, 0)),
            pl.BlockSpec((TILE, 1), lambda i: (i, 0)),
            pl.BlockSpec((D_IN, D), rep),
            pl.BlockSpec((1, D), rep),
            pl.BlockSpec((D, D), rep),
            pl.BlockSpec((1, D), rep),
            pl.BlockSpec((D, NACT), rep),
            pl.BlockSpec((1, NACT), rep),
            pl.BlockSpec((1, D), rep),
            pl.BlockSpec((1, 1), rep),
        ],
        out_specs=[
            pl.BlockSpec((TILE, NACT), lambda i: (i, 0)),
            pl.BlockSpec((TILE, 1), lambda i: (i, 0)),
            pl.BlockSpec((B, 1), rep),
        ],
        out_shape=[
            jax.ShapeDtypeStruct((N, NACT), jnp.float32),
            jax.ShapeDtypeStruct((N, 1), jnp.float32),
            jax.ShapeDtypeStruct((B, 1), jnp.float32),
        ],
        scratch_shapes=[
            pltpu.VMEM((B, D), jnp.float32),
            pltpu.VMEM((B, 1), jnp.float32),
        ],
    )(entities, batch_index_col, W_emb, b_emb, W_bb, b_bb,
      W_head, b_head, W_aux, b_aux)


def _make_sc_gather():
    info = plsc.get_sparse_core_info()
    nc, ns, nl = info.num_cores, info.num_subcores, info.num_lanes
    nw = nc * ns
    per_w = A // nw

    mesh = plsc.VectorSubcoreMesh(core_axis_name="c", subcore_axis_name="s")

    @functools.partial(
        pl.kernel,
        out_type=[
            jax.ShapeDtypeStruct((A,), jnp.float32),
            jax.ShapeDtypeStruct((A,), jnp.float32),
        ],
        mesh=mesh,
        scratch_types=[
            pltpu.VMEM((per_w,), jnp.int32),
            pltpu.VMEM((per_w,), jnp.int32),
            pltpu.VMEM((per_w,), jnp.int32),
            pltpu.VMEM((per_w,), jnp.float32),
            pltpu.VMEM((per_w,), jnp.float32),
            pltpu.SemaphoreType.DMA,
            pltpu.SemaphoreType.DMA,
        ],
    )
    def sc_k(lptbl_hbm, enttbl_hbm, aidx_hbm, pa_hbm, lp_hbm, ent_hbm,
             idx_v, pa_v, fidx_v, lp_v, ent_v, sem1, sem2):
        wid = lax.axis_index("s") * nc + lax.axis_index("c")
        base = wid * per_w
        pltpu.sync_copy(aidx_hbm.at[pl.ds(base, per_w)], idx_v)
        pltpu.sync_copy(pa_hbm.at[pl.ds(base, per_w)], pa_v)
        for j in range(per_w // nl):
            sl = pl.ds(j * nl, nl)
            fidx_v[sl] = idx_v[sl] * NACT + pa_v[sl]
        cp1 = pltpu.async_copy(lptbl_hbm.at[fidx_v], lp_v, sem1)
        cp2 = pltpu.async_copy(enttbl_hbm.at[idx_v], ent_v, sem2)
        cp1.wait()
        cp2.wait()
        pltpu.sync_copy(lp_v, lp_hbm.at[pl.ds(base, per_w)])
        pltpu.sync_copy(ent_v, ent_hbm.at[pl.ds(base, per_w)])

    return sc_k


def kernel(entities_flat, batch_index, actor_idx, prev_actions,
           W_emb, b_emb, W_bb, b_bb, W_head, b_head, W_aux, b_aux):
    bi = batch_index.astype(jnp.int32).reshape(N, 1)
    lptbl, enttbl, aux = _tc_stage(
        entities_flat, bi, W_emb, b_emb.reshape(1, D), W_bb, b_bb.reshape(1, D),
        W_head, b_head.reshape(1, NACT), W_aux.reshape(1, D), b_aux.reshape(1, 1))
    sc_k = _make_sc_gather()
    log_prob, entropy = sc_k(lptbl.reshape(N * NACT), enttbl.reshape(N),
                             actor_idx.astype(jnp.int32),
                             prev_actions.astype(jnp.int32))
    return (log_prob, entropy, aux)


# speedup vs baseline: 1.3309x; 1.3309x over previous
import functools

import jax
import jax.numpy as jnp
from jax import lax
from jax.experimental import pallas as pl
from jax.experimental.pallas import tpu as pltpu
from jax.experimental.pallas import tpu_sc as plsc

N, D_IN, D, A, B, NACT = 16384, 64, 256, 4096, 16, 16
TILE = 2048


def _tc_body(e_ref, bi_ref, w1_ref, b1_ref, w2_ref, b2_ref, wh_ref, bh_ref,
             wa_ref, ba_ref, lp_ref, ent_ref, aux_ref, seg_acc, cnt_acc):
    i = pl.program_id(0)

    emb = jnp.maximum(
        jnp.dot(e_ref[...], w1_ref[...], preferred_element_type=jnp.float32)
        + b1_ref[...], 0.0)
    x = jnp.maximum(
        jnp.dot(emb, w2_ref[...], preferred_element_type=jnp.float32)
        + b2_ref[...], 0.0)

    logits_t = lax.dot_general(wh_ref[...], x, (((1,), (1,)), ((), ())),
                               preferred_element_type=jnp.float32) + bh_ref[...]
    m = jnp.max(logits_t, axis=0, keepdims=True)
    t = logits_t - m
    s = jnp.exp(t)
    ssum = jnp.sum(s, axis=0, keepdims=True)
    logsum = jnp.log(ssum)
    lp_ref[...] = t - logsum
    ent_ref[...] = logsum - jnp.sum(s * t, axis=0, keepdims=True) / ssum

    seg_ids = lax.broadcasted_iota(jnp.int32, (TILE, B), 1)
    onehot = (bi_ref[...] == seg_ids).astype(jnp.float32)
    x_hi = x.astype(jnp.bfloat16).astype(jnp.float32)
    x_lo = x - x_hi
    dims = (((0,), (0,)), ((), ()))
    seg = (lax.dot_general(onehot, x_hi, dims,
                           preferred_element_type=jnp.float32)
           + lax.dot_general(onehot, x_lo, dims,
                             preferred_element_type=jnp.float32))
    cnt = lax.dot_general(onehot, jnp.ones((TILE, 1), jnp.float32), dims,
                          preferred_element_type=jnp.float32)

    @pl.when(i == 0)
    def _init():
        seg_acc[...] = seg
        cnt_acc[...] = cnt

    @pl.when(i > 0)
    def _accum():
        seg_acc[...] += seg
        cnt_acc[...] += cnt

    @pl.when(i == pl.num_programs(0) - 1)
    def _finalize():
        pooled = seg_acc[...] / jnp.maximum(cnt_acc[...], 1.0)
        pooled_b = pooled.astype(jnp.bfloat16).astype(jnp.float32)
        wa_b = wa_ref[...].astype(jnp.bfloat16).astype(jnp.float32)
        aux_ref[...] = (jnp.sum(pooled_b * wa_b, axis=1, keepdims=True)
                        + ba_ref[...])


def _tc_stage(entities, batch_index_col, W_emb, b_emb, W_bb, b_bb,
              W_head, b_head, W_aux, b_aux):
    grid = N // TILE
    rep = lambda i: (0, 0)
    return pl.pallas_call(
        _tc_body,
        grid=(grid,),
        in_specs=[
            pl.BlockSpec((TILE, D_IN), lambda i: (i, 0)),
            pl.BlockSpec((TILE, 1), lambda i: (i, 0)),
            pl.BlockSpec((D_IN, D), rep),
            pl.BlockSpec((1, D), rep),
            pl.BlockSpec((D, D), rep),
            pl.BlockSpec((1, D), rep),
            pl.BlockSpec((NACT, D), rep),
            pl.BlockSpec((NACT, 1), rep),
            pl.BlockSpec((1, D), rep),
            pl.BlockSpec((1, 1), rep),
        ],
        out_specs=[
            pl.BlockSpec((NACT, TILE), lambda i: (0, i)),
            pl.BlockSpec((1, TILE), lambda i: (0, i)),
            pl.BlockSpec((B, 1), rep),
        ],
        out_shape=[
            jax.ShapeDtypeStruct((NACT, N), jnp.float32),
            jax.ShapeDtypeStruct((1, N), jnp.float32),
            jax.ShapeDtypeStruct((B, 1), jnp.float32),
        ],
        scratch_shapes=[
            pltpu.VMEM((B, D), jnp.float32),
            pltpu.VMEM((B, 1), jnp.float32),
        ],
    )(entities, batch_index_col, W_emb, b_emb, W_bb, b_bb,
      W_head, b_head, W_aux, b_aux)


def _make_sc_gather():
    info = plsc.get_sparse_core_info()
    nc, ns, nl = info.num_cores, info.num_subcores, info.num_lanes
    nw = nc * ns
    per_w = A // nw

    mesh = plsc.VectorSubcoreMesh(core_axis_name="c", subcore_axis_name="s")

    @functools.partial(
        pl.kernel,
        out_type=[
            jax.ShapeDtypeStruct((A,), jnp.float32),
            jax.ShapeDtypeStruct((A,), jnp.float32),
        ],
        mesh=mesh,
        scratch_types=[
            pltpu.VMEM((per_w,), jnp.int32),
            pltpu.VMEM((per_w,), jnp.int32),
            pltpu.VMEM((per_w,), jnp.int32),
            pltpu.VMEM((per_w,), jnp.float32),
            pltpu.VMEM((per_w,), jnp.float32),
            pltpu.SemaphoreType.DMA,
            pltpu.SemaphoreType.DMA,
        ],
    )
    def sc_k(lptbl_hbm, enttbl_hbm, aidx_hbm, pa_hbm, lp_hbm, ent_hbm,
             idx_v, pa_v, fidx_v, lp_v, ent_v, sem1, sem2):
        wid = lax.axis_index("s") * nc + lax.axis_index("c")
        base = wid * per_w
        pltpu.sync_copy(aidx_hbm.at[pl.ds(base, per_w)], idx_v)
        pltpu.sync_copy(pa_hbm.at[pl.ds(base, per_w)], pa_v)
        for j in range(per_w // nl):
            sl = pl.ds(j * nl, nl)
            fidx_v[sl] = pa_v[sl] * N + idx_v[sl]
        cp1 = pltpu.async_copy(lptbl_hbm.at[fidx_v], lp_v, sem1)
        cp2 = pltpu.async_copy(enttbl_hbm.at[idx_v], ent_v, sem2)
        cp1.wait()
        cp2.wait()
        pltpu.sync_copy(lp_v, lp_hbm.at[pl.ds(base, per_w)])
        pltpu.sync_copy(ent_v, ent_hbm.at[pl.ds(base, per_w)])

    return sc_k


def kernel(entities_flat, batch_index, actor_idx, prev_actions,
           W_emb, b_emb, W_bb, b_bb, W_head, b_head, W_aux, b_aux):
    bi = batch_index.astype(jnp.int32).reshape(N, 1)
    lptbl, enttbl, aux = _tc_stage(
        entities_flat, bi, W_emb, b_emb.reshape(1, D), W_bb, b_bb.reshape(1, D),
        W_head.T, b_head.reshape(NACT, 1), W_aux.reshape(1, D),
        b_aux.reshape(1, 1))
    sc_k = _make_sc_gather()
    log_prob, entropy = sc_k(lptbl.reshape(NACT * N), enttbl.reshape(N),
                             actor_idx.astype(jnp.int32),
                             prev_actions.astype(jnp.int32))
    return (log_prob, entropy, aux)


# speedup vs baseline: 1.4888x; 1.1186x over previous
import functools

import jax
import jax.numpy as jnp
from jax import lax
from jax.experimental import pallas as pl
from jax.experimental.pallas import tpu as pltpu
from jax.experimental.pallas import tpu_sc as plsc

N, D_IN, D, A, B, NACT = 16384, 64, 256, 4096, 16, 16
TILE = 2048


def _tc_body(e_ref, bi_ref, w1_ref, b1_ref, w2_ref, b2_ref, wh_ref, bh_ref,
             wa_ref, ba_ref, lp_ref, ent_ref, aux_ref, seg_acc, cnt_acc):
    i = pl.program_id(0)

    emb = jnp.maximum(
        jnp.dot(e_ref[...], w1_ref[...], preferred_element_type=jnp.float32)
        + b1_ref[...], 0.0)
    x = jnp.maximum(
        jnp.dot(emb, w2_ref[...], preferred_element_type=jnp.float32)
        + b2_ref[...], 0.0)

    logits_t = lax.dot_general(wh_ref[...], x, (((1,), (1,)), ((), ())),
                               preferred_element_type=jnp.float32) + bh_ref[...]
    m = jnp.max(logits_t, axis=0, keepdims=True)
    t = logits_t - m
    s = jnp.exp(t)
    ssum = jnp.sum(s, axis=0, keepdims=True)
    logsum = jnp.log(ssum)
    lp_ref[...] = t - logsum
    ent_ref[...] = logsum - jnp.sum(s * t, axis=0, keepdims=True) / ssum

    seg_ids = lax.broadcasted_iota(jnp.int32, (B, TILE), 0)
    onehot_t = (bi_ref[...] == seg_ids).astype(jnp.float32)
    x_hi = x.astype(jnp.bfloat16).astype(jnp.float32)
    x_lo = x - x_hi
    seg = (jnp.dot(onehot_t, x_hi, preferred_element_type=jnp.float32)
           + jnp.dot(onehot_t, x_lo, preferred_element_type=jnp.float32))
    cnt = jnp.sum(onehot_t, axis=1, keepdims=True)

    @pl.when(i == 0)
    def _init():
        seg_acc[...] = seg
        cnt_acc[...] = cnt

    @pl.when(i > 0)
    def _accum():
        seg_acc[...] += seg
        cnt_acc[...] += cnt

    @pl.when(i == pl.num_programs(0) - 1)
    def _finalize():
        pooled = seg_acc[...] / jnp.maximum(cnt_acc[...], 1.0)
        pooled_b = pooled.astype(jnp.bfloat16).astype(jnp.float32)
        wa_b = wa_ref[...].astype(jnp.bfloat16).astype(jnp.float32)
        aux_ref[...] = (jnp.sum(pooled_b * wa_b, axis=1, keepdims=True)
                        + ba_ref[...])


def _tc_stage(entities, batch_index_col, W_emb, b_emb, W_bb, b_bb,
              W_head, b_head, W_aux, b_aux):
    grid = N // TILE
    rep = lambda i: (0, 0)
    return pl.pallas_call(
        _tc_body,
        grid=(grid,),
        in_specs=[
            pl.BlockSpec((TILE, D_IN), lambda i: (i, 0)),
            pl.BlockSpec((1, TILE), lambda i: (0, i)),
            pl.BlockSpec((D_IN, D), rep),
            pl.BlockSpec((1, D), rep),
            pl.BlockSpec((D, D), rep),
            pl.BlockSpec((1, D), rep),
            pl.BlockSpec((NACT, D), rep),
            pl.BlockSpec((NACT, 1), rep),
            pl.BlockSpec((1, D), rep),
            pl.BlockSpec((1, 1), rep),
        ],
        out_specs=[
            pl.BlockSpec((NACT, TILE), lambda i: (0, i)),
            pl.BlockSpec((1, TILE), lambda i: (0, i)),
            pl.BlockSpec((B, 1), rep),
        ],
        out_shape=[
            jax.ShapeDtypeStruct((NACT, N), jnp.float32),
            jax.ShapeDtypeStruct((1, N), jnp.float32),
            jax.ShapeDtypeStruct((B, 1), jnp.float32),
        ],
        scratch_shapes=[
            pltpu.VMEM((B, D), jnp.float32),
            pltpu.VMEM((B, 1), jnp.float32),
        ],
    )(entities, batch_index_col, W_emb, b_emb, W_bb, b_bb,
      W_head, b_head, W_aux, b_aux)


def _make_sc_gather():
    info = plsc.get_sparse_core_info()
    nc, ns, nl = info.num_cores, info.num_subcores, info.num_lanes
    nw = nc * ns
    per_w = A // nw

    mesh = plsc.VectorSubcoreMesh(core_axis_name="c", subcore_axis_name="s")

    @functools.partial(
        pl.kernel,
        out_type=[
            jax.ShapeDtypeStruct((A,), jnp.float32),
            jax.ShapeDtypeStruct((A,), jnp.float32),
        ],
        mesh=mesh,
        scratch_types=[
            pltpu.VMEM((per_w,), jnp.int32),
            pltpu.VMEM((per_w,), jnp.int32),
            pltpu.VMEM((per_w,), jnp.int32),
            pltpu.VMEM((per_w,), jnp.float32),
            pltpu.VMEM((per_w,), jnp.float32),
            pltpu.SemaphoreType.DMA,
            pltpu.SemaphoreType.DMA,
        ],
    )
    def sc_k(lptbl_hbm, enttbl_hbm, aidx_hbm, pa_hbm, lp_hbm, ent_hbm,
             idx_v, pa_v, fidx_v, lp_v, ent_v, sem1, sem2):
        wid = lax.axis_index("s") * nc + lax.axis_index("c")
        base = wid * per_w
        pltpu.sync_copy(aidx_hbm.at[pl.ds(base, per_w)], idx_v)
        pltpu.sync_copy(pa_hbm.at[pl.ds(base, per_w)], pa_v)
        for j in range(per_w // nl):
            sl = pl.ds(j * nl, nl)
            fidx_v[sl] = pa_v[sl] * N + idx_v[sl]
        cp1 = pltpu.async_copy(lptbl_hbm.at[fidx_v], lp_v, sem1)
        cp2 = pltpu.async_copy(enttbl_hbm.at[idx_v], ent_v, sem2)
        cp1.wait()
        cp2.wait()
        pltpu.sync_copy(lp_v, lp_hbm.at[pl.ds(base, per_w)])
        pltpu.sync_copy(ent_v, ent_hbm.at[pl.ds(base, per_w)])

    return sc_k


def kernel(entities_flat, batch_index, actor_idx, prev_actions,
           W_emb, b_emb, W_bb, b_bb, W_head, b_head, W_aux, b_aux):
    bi = batch_index.astype(jnp.int32).reshape(1, N)
    lptbl, enttbl, aux = _tc_stage(
        entities_flat, bi, W_emb, b_emb.reshape(1, D), W_bb, b_bb.reshape(1, D),
        W_head.T, b_head.reshape(NACT, 1), W_aux.reshape(1, D),
        b_aux.reshape(1, 1))
    sc_k = _make_sc_gather()
    log_prob, entropy = sc_k(lptbl.reshape(NACT * N), enttbl.reshape(N),
                             actor_idx.astype(jnp.int32),
                             prev_actions.astype(jnp.int32))
    return (log_prob, entropy, aux)


# speedup vs baseline: 1.5070x; 1.0123x over previous
import functools

import jax
import jax.numpy as jnp
from jax import lax
from jax.experimental import pallas as pl
from jax.experimental.pallas import tpu as pltpu
from jax.experimental.pallas import tpu_sc as plsc

N, D_IN, D, A, B, NACT = 16384, 64, 256, 4096, 16, 16
TILE = 2048


def _tc_body(e_ref, bi_ref, w1_ref, b1_ref, w2_ref, b2_ref, wh_ref, bh_ref,
             wa_ref, ba_ref, lp_ref, ent_ref, aux_ref, seg_acc, cnt_acc):
    i = pl.program_id(0)

    emb = jnp.maximum(
        jnp.dot(e_ref[...], w1_ref[...], preferred_element_type=jnp.float32)
        + b1_ref[...], 0.0)
    x = jnp.maximum(
        jnp.dot(emb, w2_ref[...], preferred_element_type=jnp.float32)
        + b2_ref[...], 0.0)

    logits_t = lax.dot_general(wh_ref[...], x, (((1,), (1,)), ((), ())),
                               preferred_element_type=jnp.float32) + bh_ref[...]
    m = jnp.max(logits_t, axis=0, keepdims=True)
    t = logits_t - m
    s = jnp.exp(t)
    ssum = jnp.sum(s, axis=0, keepdims=True)
    logsum = jnp.log(ssum)
    lp_ref[...] = t - logsum
    ent_ref[...] = logsum - jnp.sum(s * t, axis=0, keepdims=True) / ssum

    seg_ids = lax.broadcasted_iota(jnp.int32, (B, TILE), 0)
    onehot_t = (bi_ref[...] == seg_ids).astype(jnp.float32)
    x_hi = x.astype(jnp.bfloat16).astype(jnp.float32)
    x_lo = x - x_hi
    seg = (jnp.dot(onehot_t, x_hi, preferred_element_type=jnp.float32)
           + jnp.dot(onehot_t, x_lo, preferred_element_type=jnp.float32))
    cnt = jnp.sum(onehot_t, axis=1, keepdims=True)

    @pl.when(i == 0)
    def _init():
        seg_acc[...] = seg
        cnt_acc[...] = cnt

    @pl.when(i > 0)
    def _accum():
        seg_acc[...] += seg
        cnt_acc[...] += cnt

    @pl.when(i == pl.num_programs(0) - 1)
    def _finalize():
        pooled = seg_acc[...] / jnp.maximum(cnt_acc[...], 1.0)
        pooled_b = pooled.astype(jnp.bfloat16).astype(jnp.float32)
        wa_b = wa_ref[...].astype(jnp.bfloat16).astype(jnp.float32)
        aux_ref[...] = (jnp.sum(pooled_b * wa_b, axis=1, keepdims=True)
                        + ba_ref[...])


def _tc_stage(entities, batch_index_col, W_emb, b_emb, W_bb, b_bb,
              W_head, b_head, W_aux, b_aux):
    grid = N // TILE
    rep = lambda i: (0, 0)
    return pl.pallas_call(
        _tc_body,
        grid=(grid,),
        in_specs=[
            pl.BlockSpec((TILE, D_IN), lambda i: (i, 0)),
            pl.BlockSpec((1, TILE), lambda i: (0, i)),
            pl.BlockSpec((D_IN, D), rep),
            pl.BlockSpec((1, D), rep),
            pl.BlockSpec((D, D), rep),
            pl.BlockSpec((1, D), rep),
            pl.BlockSpec((NACT, D), rep),
            pl.BlockSpec((NACT, 1), rep),
            pl.BlockSpec((1, D), rep),
            pl.BlockSpec((1, 1), rep),
        ],
        out_specs=[
            pl.BlockSpec((NACT, TILE), lambda i: (0, i)),
            pl.BlockSpec((1, TILE), lambda i: (0, i)),
            pl.BlockSpec((B, 1), rep),
        ],
        out_shape=[
            jax.ShapeDtypeStruct((NACT, N), jnp.float32),
            jax.ShapeDtypeStruct((1, N), jnp.float32),
            jax.ShapeDtypeStruct((B, 1), jnp.float32),
        ],
        scratch_shapes=[
            pltpu.VMEM((B, D), jnp.float32),
            pltpu.VMEM((B, 1), jnp.float32),
        ],
    )(entities, batch_index_col, W_emb, b_emb, W_bb, b_bb,
      W_head, b_head, W_aux, b_aux)


def _make_sc_gather():
    info = plsc.get_sparse_core_info()
    nc, ns, nl = info.num_cores, info.num_subcores, info.num_lanes
    nw = nc * ns
    per_w = A // nw

    mesh = plsc.VectorSubcoreMesh(core_axis_name="c", subcore_axis_name="s")

    @functools.partial(
        pl.kernel,
        out_type=[
            jax.ShapeDtypeStruct((A,), jnp.float32),
            jax.ShapeDtypeStruct((A,), jnp.float32),
        ],
        mesh=mesh,
        scratch_types=[
            pltpu.VMEM((per_w,), jnp.int32),
            pltpu.VMEM((per_w,), jnp.int32),
            pltpu.VMEM((per_w,), jnp.int32),
            pltpu.VMEM((per_w,), jnp.float32),
            pltpu.VMEM((per_w,), jnp.float32),
            pltpu.SemaphoreType.DMA,
            pltpu.SemaphoreType.DMA,
        ],
    )
    def sc_k(lptbl_hbm, enttbl_hbm, aidx_hbm, pa_hbm, lp_hbm, ent_hbm,
             idx_v, pa_v, fidx_v, lp_v, ent_v, sem1, sem2):
        wid = lax.axis_index("s") * nc + lax.axis_index("c")
        base = wid * per_w
        cpa = pltpu.async_copy(aidx_hbm.at[pl.ds(base, per_w)], idx_v, sem1)
        cpb = pltpu.async_copy(pa_hbm.at[pl.ds(base, per_w)], pa_v, sem2)
        cpa.wait()
        cpb.wait()
        for j in range(per_w // nl):
            sl = pl.ds(j * nl, nl)
            fidx_v[sl] = pa_v[sl] * N + idx_v[sl]
        cp1 = pltpu.async_copy(lptbl_hbm.at[fidx_v], lp_v, sem1)
        cp2 = pltpu.async_copy(enttbl_hbm.at[idx_v], ent_v, sem2)
        cp1.wait()
        cp2.wait()
        pltpu.sync_copy(lp_v, lp_hbm.at[pl.ds(base, per_w)])
        pltpu.sync_copy(ent_v, ent_hbm.at[pl.ds(base, per_w)])

    return sc_k


def kernel(entities_flat, batch_index, actor_idx, prev_actions,
           W_emb, b_emb, W_bb, b_bb, W_head, b_head, W_aux, b_aux):
    bi = batch_index.astype(jnp.int32).reshape(1, N)
    lptbl, enttbl, aux = _tc_stage(
        entities_flat, bi, W_emb, b_emb.reshape(1, D), W_bb, b_bb.reshape(1, D),
        W_head.T, b_head.reshape(NACT, 1), W_aux.reshape(1, D),
        b_aux.reshape(1, 1))
    sc_k = _make_sc_gather()
    log_prob, entropy = sc_k(lptbl.reshape(NACT * N), enttbl.reshape(N),
                             actor_idx.astype(jnp.int32),
                             prev_actions.astype(jnp.int32))
    return (log_prob, entropy, aux)


# speedup vs baseline: 1.5435x; 1.0242x over previous
import functools

import jax
import jax.numpy as jnp
from jax import lax
from jax.experimental import pallas as pl
from jax.experimental.pallas import tpu as pltpu
from jax.experimental.pallas import tpu_sc as plsc

N, D_IN, D, A, B, NACT = 16384, 64, 256, 4096, 16, 16
TILE = 2048


def _tc_body(e_ref, bi_ref, w1_ref, b1_ref, w2_ref, b2_ref, wh_ref, bh_ref,
             wa_ref, ba_ref, lp_ref, ent_ref, aux_ref, seg_acc, cnt_acc):
    i = pl.program_id(0)

    emb = jnp.maximum(
        jnp.dot(e_ref[...], w1_ref[...], preferred_element_type=jnp.float32)
        + b1_ref[...], 0.0)
    x = jnp.maximum(
        jnp.dot(emb, w2_ref[...], preferred_element_type=jnp.float32)
        + b2_ref[...], 0.0)

    logits_t = lax.dot_general(wh_ref[...], x, (((1,), (1,)), ((), ())),
                               preferred_element_type=jnp.float32) + bh_ref[...]
    m = jnp.max(logits_t, axis=0, keepdims=True)
    t = logits_t - m
    s = jnp.exp(t)
    ssum = jnp.sum(s, axis=0, keepdims=True)
    logsum = jnp.log(ssum)
    lp_ref[...] = t - logsum
    ent_ref[...] = logsum - jnp.sum(s * t, axis=0, keepdims=True) / ssum

    seg_ids = lax.broadcasted_iota(jnp.int32, (B, TILE), 0)
    onehot_t = (bi_ref[...] == seg_ids).astype(jnp.float32)
    x_hi = x.astype(jnp.bfloat16).astype(jnp.float32)
    x_lo = x - x_hi
    seg = (jnp.dot(onehot_t, x_hi, preferred_element_type=jnp.float32)
           + jnp.dot(onehot_t, x_lo, preferred_element_type=jnp.float32))
    cnt = jnp.sum(onehot_t, axis=1, keepdims=True)

    @pl.when(i == 0)
    def _init():
        seg_acc[...] = seg
        cnt_acc[...] = cnt

    @pl.when(i > 0)
    def _accum():
        seg_acc[...] += seg
        cnt_acc[...] += cnt

    @pl.when(i == pl.num_programs(0) - 1)
    def _finalize():
        pooled = seg_acc[...] / jnp.maximum(cnt_acc[...], 1.0)
        pooled_b = pooled.astype(jnp.bfloat16).astype(jnp.float32)
        wa_b = wa_ref[...].astype(jnp.bfloat16).astype(jnp.float32)
        aux_ref[...] = (jnp.sum(pooled_b * wa_b, axis=1, keepdims=True)
                        + ba_ref[...])


def _tc_stage(entities, batch_index_col, W_emb, b_emb, W_bb, b_bb,
              W_head, b_head, W_aux, b_aux):
    grid = N // TILE
    rep = lambda i: (0, 0)
    return pl.pallas_call(
        _tc_body,
        grid=(grid,),
        in_specs=[
            pl.BlockSpec((TILE, D_IN), lambda i: (i, 0)),
            pl.BlockSpec((1, TILE), lambda i: (0, i)),
            pl.BlockSpec((D_IN, D), rep),
            pl.BlockSpec((1, D), rep),
            pl.BlockSpec((D, D), rep),
            pl.BlockSpec((1, D), rep),
            pl.BlockSpec((NACT, D), rep),
            pl.BlockSpec((NACT, 1), rep),
            pl.BlockSpec((1, D), rep),
            pl.BlockSpec((1, 1), rep),
        ],
        out_specs=[
            pl.BlockSpec((NACT, TILE), lambda i: (0, i)),
            pl.BlockSpec((1, TILE), lambda i: (0, i)),
            pl.BlockSpec((B, 1), rep),
        ],
        out_shape=[
            jax.ShapeDtypeStruct((NACT, N), jnp.float32),
            jax.ShapeDtypeStruct((1, N), jnp.float32),
            jax.ShapeDtypeStruct((B, 1), jnp.float32),
        ],
        scratch_shapes=[
            pltpu.VMEM((B, D), jnp.float32),
            pltpu.VMEM((B, 1), jnp.float32),
        ],
    )(entities, batch_index_col, W_emb, b_emb, W_bb, b_bb,
      W_head, b_head, W_aux, b_aux)


def _make_sc_gather():
    info = plsc.get_sparse_core_info()
    nc, ns, nl = info.num_cores, info.num_subcores, info.num_lanes
    nw = nc * ns
    per_w = A // nw

    nc = 1
    nw = nc * ns
    per_w = A // nw
    mesh = plsc.VectorSubcoreMesh(core_axis_name="c", subcore_axis_name="s",
                                  num_cores=nc)

    @functools.partial(
        pl.kernel,
        out_type=[
            jax.ShapeDtypeStruct((A,), jnp.float32),
            jax.ShapeDtypeStruct((A,), jnp.float32),
        ],
        mesh=mesh,
        scratch_types=[
            pltpu.VMEM((per_w,), jnp.int32),
            pltpu.VMEM((per_w,), jnp.int32),
            pltpu.VMEM((per_w,), jnp.int32),
            pltpu.VMEM((per_w,), jnp.float32),
            pltpu.VMEM((per_w,), jnp.float32),
            pltpu.SemaphoreType.DMA,
            pltpu.SemaphoreType.DMA,
        ],
    )
    def sc_k(lptbl_hbm, enttbl_hbm, aidx_hbm, pa_hbm, lp_hbm, ent_hbm,
             idx_v, pa_v, fidx_v, lp_v, ent_v, sem1, sem2):
        wid = lax.axis_index("s") * nc + lax.axis_index("c")
        base = wid * per_w
        cpa = pltpu.async_copy(aidx_hbm.at[pl.ds(base, per_w)], idx_v, sem1)
        cpb = pltpu.async_copy(pa_hbm.at[pl.ds(base, per_w)], pa_v, sem2)
        cpa.wait()
        cpb.wait()
        for j in range(per_w // nl):
            sl = pl.ds(j * nl, nl)
            fidx_v[sl] = pa_v[sl] * N + idx_v[sl]
        cp1 = pltpu.async_copy(lptbl_hbm.at[fidx_v], lp_v, sem1)
        cp2 = pltpu.async_copy(enttbl_hbm.at[idx_v], ent_v, sem2)
        cp1.wait()
        cp2.wait()
        pltpu.sync_copy(lp_v, lp_hbm.at[pl.ds(base, per_w)])
        pltpu.sync_copy(ent_v, ent_hbm.at[pl.ds(base, per_w)])

    return sc_k


def kernel(entities_flat, batch_index, actor_idx, prev_actions,
           W_emb, b_emb, W_bb, b_bb, W_head, b_head, W_aux, b_aux):
    bi = batch_index.astype(jnp.int32).reshape(1, N)
    lptbl, enttbl, aux = _tc_stage(
        entities_flat, bi, W_emb, b_emb.reshape(1, D), W_bb, b_bb.reshape(1, D),
        W_head.T, b_head.reshape(NACT, 1), W_aux.reshape(1, D),
        b_aux.reshape(1, 1))
    sc_k = _make_sc_gather()
    log_prob, entropy = sc_k(lptbl.reshape(NACT * N), enttbl.reshape(N),
                             actor_idx.astype(jnp.int32),
                             prev_actions.astype(jnp.int32))
    return (log_prob, entropy, aux)


# speedup vs baseline: 1.6233x; 1.0517x over previous
import functools

import jax
import jax.numpy as jnp
from jax import lax
from jax.experimental import pallas as pl
from jax.experimental.pallas import tpu as pltpu
from jax.experimental.pallas import tpu_sc as plsc

N, D_IN, D, A, B, NACT = 16384, 64, 256, 4096, 16, 16
TILE = 2048


def _tc_body(e_ref, bi_ref, w1_ref, b1_ref, w2_ref, b2_ref, wh_ref, bh_ref,
             wa_ref, ba_ref, lp_ref, ent_ref, aux_ref, seg_acc, cnt_acc):
    i = pl.program_id(0)

    emb = jnp.maximum(
        jnp.dot(e_ref[...], w1_ref[...], preferred_element_type=jnp.float32)
        + b1_ref[...], 0.0)
    x = jnp.maximum(
        jnp.dot(emb, w2_ref[...], preferred_element_type=jnp.float32)
        + b2_ref[...], 0.0)

    seg_ids = lax.broadcasted_iota(jnp.int32, (B, TILE), 0)
    onehot_t = (bi_ref[...] == seg_ids).astype(jnp.float32)
    seg = jnp.dot(onehot_t, x, preferred_element_type=jnp.float32)
    cnt = jnp.sum(onehot_t, axis=1, keepdims=True)

    logits_t = lax.dot_general(wh_ref[...], x, (((1,), (1,)), ((), ())),
                               preferred_element_type=jnp.float32) + bh_ref[...]
    m = jnp.max(logits_t, axis=0, keepdims=True)
    t = logits_t - m
    s = jnp.exp(t)
    ssum = jnp.sum(s, axis=0, keepdims=True)
    logsum = jnp.log(ssum)
    lp_ref[...] = t - logsum
    ent_ref[...] = logsum - jnp.sum(s * t, axis=0, keepdims=True) / ssum

    @pl.when(i == 0)
    def _init():
        seg_acc[...] = seg
        cnt_acc[...] = cnt

    @pl.when(i > 0)
    def _accum():
        seg_acc[...] += seg
        cnt_acc[...] += cnt

    @pl.when(i == pl.num_programs(0) - 1)
    def _finalize():
        pooled = seg_acc[...] / jnp.maximum(cnt_acc[...], 1.0)
        pooled_b = pooled.astype(jnp.bfloat16).astype(jnp.float32)
        wa_b = wa_ref[...].astype(jnp.bfloat16).astype(jnp.float32)
        aux_ref[...] = (jnp.sum(pooled_b * wa_b, axis=1, keepdims=True)
                        + ba_ref[...])


def _tc_stage(entities, batch_index_col, W_emb, b_emb, W_bb, b_bb,
              W_head, b_head, W_aux, b_aux):
    grid = N // TILE
    rep = lambda i: (0, 0)
    return pl.pallas_call(
        _tc_body,
        grid=(grid,),
        in_specs=[
            pl.BlockSpec((TILE, D_IN), lambda i: (i, 0)),
            pl.BlockSpec((1, TILE), lambda i: (0, i)),
            pl.BlockSpec((D_IN, D), rep),
            pl.BlockSpec((1, D), rep),
            pl.BlockSpec((D, D), rep),
            pl.BlockSpec((1, D), rep),
            pl.BlockSpec((NACT, D), rep),
            pl.BlockSpec((NACT, 1), rep),
            pl.BlockSpec((1, D), rep),
            pl.BlockSpec((1, 1), rep),
        ],
        out_specs=[
            pl.BlockSpec((NACT, TILE), lambda i: (0, i)),
            pl.BlockSpec((1, TILE), lambda i: (0, i)),
            pl.BlockSpec((B, 1), rep),
        ],
        out_shape=[
            jax.ShapeDtypeStruct((NACT, N), jnp.float32),
            jax.ShapeDtypeStruct((1, N), jnp.float32),
            jax.ShapeDtypeStruct((B, 1), jnp.float32),
        ],
        scratch_shapes=[
            pltpu.VMEM((B, D), jnp.float32),
            pltpu.VMEM((B, 1), jnp.float32),
        ],
    )(entities, batch_index_col, W_emb, b_emb, W_bb, b_bb,
      W_head, b_head, W_aux, b_aux)


def _make_sc_gather():
    info = plsc.get_sparse_core_info()
    nc, ns, nl = info.num_cores, info.num_subcores, info.num_lanes
    nw = nc * ns
    per_w = A // nw

    nc = 1
    nw = nc * ns
    per_w = A // nw
    mesh = plsc.VectorSubcoreMesh(core_axis_name="c", subcore_axis_name="s",
                                  num_cores=nc)

    @functools.partial(
        pl.kernel,
        out_type=[
            jax.ShapeDtypeStruct((A,), jnp.float32),
            jax.ShapeDtypeStruct((A,), jnp.float32),
        ],
        mesh=mesh,
        scratch_types=[
            pltpu.VMEM((per_w,), jnp.int32),
            pltpu.VMEM((per_w,), jnp.int32),
            pltpu.VMEM((per_w,), jnp.int32),
            pltpu.VMEM((per_w,), jnp.float32),
            pltpu.VMEM((per_w,), jnp.float32),
            pltpu.SemaphoreType.DMA,
            pltpu.SemaphoreType.DMA,
        ],
    )
    def sc_k(lptbl_hbm, enttbl_hbm, aidx_hbm, pa_hbm, lp_hbm, ent_hbm,
             idx_v, pa_v, fidx_v, lp_v, ent_v, sem1, sem2):
        wid = lax.axis_index("s") * nc + lax.axis_index("c")
        base = wid * per_w
        cpa = pltpu.async_copy(aidx_hbm.at[pl.ds(base, per_w)], idx_v, sem1)
        cpb = pltpu.async_copy(pa_hbm.at[pl.ds(base, per_w)], pa_v, sem2)
        cpa.wait()
        cpb.wait()
        for j in range(per_w // nl):
            sl = pl.ds(j * nl, nl)
            fidx_v[sl] = pa_v[sl] * N + idx_v[sl]
        cp1 = pltpu.async_copy(lptbl_hbm.at[fidx_v], lp_v, sem1)
        cp2 = pltpu.async_copy(enttbl_hbm.at[idx_v], ent_v, sem2)
        cp1.wait()
        cp2.wait()
        pltpu.sync_copy(lp_v, lp_hbm.at[pl.ds(base, per_w)])
        pltpu.sync_copy(ent_v, ent_hbm.at[pl.ds(base, per_w)])

    return sc_k


def kernel(entities_flat, batch_index, actor_idx, prev_actions,
           W_emb, b_emb, W_bb, b_bb, W_head, b_head, W_aux, b_aux):
    bi = batch_index.astype(jnp.int32).reshape(1, N)
    lptbl, enttbl, aux = _tc_stage(
        entities_flat, bi, W_emb, b_emb.reshape(1, D), W_bb, b_bb.reshape(1, D),
        W_head.T, b_head.reshape(NACT, 1), W_aux.reshape(1, D),
        b_aux.reshape(1, 1))
    sc_k = _make_sc_gather()
    log_prob, entropy = sc_k(lptbl.reshape(NACT * N), enttbl.reshape(N),
                             actor_idx.astype(jnp.int32),
                             prev_actions.astype(jnp.int32))
    return (log_prob, entropy, aux)


# speedup vs baseline: 1.6688x; 1.0281x over previous
import functools

import jax
import jax.numpy as jnp
from jax import lax
from jax.experimental import pallas as pl
from jax.experimental.pallas import tpu as pltpu
from jax.experimental.pallas import tpu_sc as plsc

N, D_IN, D, A, B, NACT = 16384, 64, 256, 4096, 16, 16
TILE = 4096


def _tc_body(e_ref, bi_ref, w1_ref, b1_ref, w2_ref, b2_ref, wh_ref, bh_ref,
             wa_ref, ba_ref, lp_ref, ent_ref, aux_ref, seg_acc, cnt_acc):
    i = pl.program_id(0)

    emb = jnp.maximum(
        jnp.dot(e_ref[...], w1_ref[...], preferred_element_type=jnp.float32)
        + b1_ref[...], 0.0)
    x = jnp.maximum(
        jnp.dot(emb, w2_ref[...], preferred_element_type=jnp.float32)
        + b2_ref[...], 0.0)

    seg_ids = lax.broadcasted_iota(jnp.int32, (B, TILE), 0)
    onehot_t = (bi_ref[...] == seg_ids).astype(jnp.float32)
    seg = jnp.dot(onehot_t, x, preferred_element_type=jnp.float32)
    cnt = jnp.sum(onehot_t, axis=1, keepdims=True)

    logits_t = lax.dot_general(wh_ref[...], x, (((1,), (1,)), ((), ())),
                               preferred_element_type=jnp.float32) + bh_ref[...]
    m = jnp.max(logits_t, axis=0, keepdims=True)
    t = logits_t - m
    s = jnp.exp(t)
    ssum = jnp.sum(s, axis=0, keepdims=True)
    logsum = jnp.log(ssum)
    lp_ref[...] = t - logsum
    ent_ref[...] = logsum - jnp.sum(s * t, axis=0, keepdims=True) / ssum

    @pl.when(i == 0)
    def _init():
        seg_acc[...] = seg
        cnt_acc[...] = cnt

    @pl.when(i > 0)
    def _accum():
        seg_acc[...] += seg
        cnt_acc[...] += cnt

    @pl.when(i == pl.num_programs(0) - 1)
    def _finalize():
        pooled = seg_acc[...] / jnp.maximum(cnt_acc[...], 1.0)
        pooled_b = pooled.astype(jnp.bfloat16).astype(jnp.float32)
        wa_b = wa_ref[...].astype(jnp.bfloat16).astype(jnp.float32)
        aux_ref[...] = (jnp.sum(pooled_b * wa_b, axis=1, keepdims=True)
                        + ba_ref[...])


def _tc_stage(entities, batch_index_col, W_emb, b_emb, W_bb, b_bb,
              W_head, b_head, W_aux, b_aux):
    grid = N // TILE
    rep = lambda i: (0, 0)
    return pl.pallas_call(
        _tc_body,
        grid=(grid,),
        in_specs=[
            pl.BlockSpec((TILE, D_IN), lambda i: (i, 0)),
            pl.BlockSpec((1, TILE), lambda i: (0, i)),
            pl.BlockSpec((D_IN, D), rep),
            pl.BlockSpec((1, D), rep),
            pl.BlockSpec((D, D), rep),
            pl.BlockSpec((1, D), rep),
            pl.BlockSpec((NACT, D), rep),
            pl.BlockSpec((NACT, 1), rep),
            pl.BlockSpec((1, D), rep),
            pl.BlockSpec((1, 1), rep),
        ],
        out_specs=[
            pl.BlockSpec((NACT, TILE), lambda i: (0, i)),
            pl.BlockSpec((1, TILE), lambda i: (0, i)),
            pl.BlockSpec((B, 1), rep),
        ],
        out_shape=[
            jax.ShapeDtypeStruct((NACT, N), jnp.float32),
            jax.ShapeDtypeStruct((1, N), jnp.float32),
            jax.ShapeDtypeStruct((B, 1), jnp.float32),
        ],
        scratch_shapes=[
            pltpu.VMEM((B, D), jnp.float32),
            pltpu.VMEM((B, 1), jnp.float32),
        ],
    )(entities, batch_index_col, W_emb, b_emb, W_bb, b_bb,
      W_head, b_head, W_aux, b_aux)


def _make_sc_gather():
    info = plsc.get_sparse_core_info()
    nc, ns, nl = info.num_cores, info.num_subcores, info.num_lanes
    nw = nc * ns
    per_w = A // nw

    nc = 1
    nw = nc * ns
    per_w = A // nw
    mesh = plsc.VectorSubcoreMesh(core_axis_name="c", subcore_axis_name="s",
                                  num_cores=nc)

    @functools.partial(
        pl.kernel,
        out_type=[
            jax.ShapeDtypeStruct((A,), jnp.float32),
            jax.ShapeDtypeStruct((A,), jnp.float32),
        ],
        mesh=mesh,
        scratch_types=[
            pltpu.VMEM((per_w,), jnp.int32),
            pltpu.VMEM((per_w,), jnp.int32),
            pltpu.VMEM((per_w,), jnp.int32),
            pltpu.VMEM((per_w,), jnp.float32),
            pltpu.VMEM((per_w,), jnp.float32),
            pltpu.SemaphoreType.DMA,
            pltpu.SemaphoreType.DMA,
        ],
    )
    def sc_k(lptbl_hbm, enttbl_hbm, aidx_hbm, pa_hbm, lp_hbm, ent_hbm,
             idx_v, pa_v, fidx_v, lp_v, ent_v, sem1, sem2):
        wid = lax.axis_index("s") * nc + lax.axis_index("c")
        base = wid * per_w
        cpa = pltpu.async_copy(aidx_hbm.at[pl.ds(base, per_w)], idx_v, sem1)
        cpb = pltpu.async_copy(pa_hbm.at[pl.ds(base, per_w)], pa_v, sem2)
        cpa.wait()
        cpb.wait()
        for j in range(per_w // nl):
            sl = pl.ds(j * nl, nl)
            fidx_v[sl] = pa_v[sl] * N + idx_v[sl]
        cp1 = pltpu.async_copy(lptbl_hbm.at[fidx_v], lp_v, sem1)
        cp2 = pltpu.async_copy(enttbl_hbm.at[idx_v], ent_v, sem2)
        cp1.wait()
        cp2.wait()
        pltpu.sync_copy(lp_v, lp_hbm.at[pl.ds(base, per_w)])
        pltpu.sync_copy(ent_v, ent_hbm.at[pl.ds(base, per_w)])

    return sc_k


def kernel(entities_flat, batch_index, actor_idx, prev_actions,
           W_emb, b_emb, W_bb, b_bb, W_head, b_head, W_aux, b_aux):
    bi = batch_index.astype(jnp.int32).reshape(1, N)
    lptbl, enttbl, aux = _tc_stage(
        entities_flat, bi, W_emb, b_emb.reshape(1, D), W_bb, b_bb.reshape(1, D),
        W_head.T, b_head.reshape(NACT, 1), W_aux.reshape(1, D),
        b_aux.reshape(1, 1))
    sc_k = _make_sc_gather()
    log_prob, entropy = sc_k(lptbl.reshape(NACT * N), enttbl.reshape(N),
                             actor_idx.astype(jnp.int32),
                             prev_actions.astype(jnp.int32))
    return (log_prob, entropy, aux)
